# baseline (speedup 1.0000x reference)
import jax
import jax.numpy as jnp
from jax.experimental import pallas as pl


def _tail(scat_ref, x_ref, o_ref):
    o_ref[...] = 4.0 * jnp.tanh(scat_ref[...] + 0.1 * x_ref[...])


def kernel(x, edge_index, edge_attr, W1, b1, W2, b2):
    row = edge_index[0]
    col = edge_index[1]
    x_i = jnp.take(x, row, axis=0)
    x_j = jnp.take(x, col, axis=0)
    rel_pos = x_j - x_i
    direction = 2.3094 * jnp.tanh(rel_pos)
    message_input = jnp.concatenate([x_i, x_j, edge_attr, direction], axis=-1)
    h = jax.nn.relu(message_input @ W1 + b1)
    messages = h @ W2 + b2
    scat = jnp.zeros_like(x).at[col].max(messages)
    N, D = x.shape
    BN = 1000
    out = pl.pallas_call(
        _tail,
        grid=(N // BN,),
        in_specs=[
            pl.BlockSpec((BN, D), lambda i: (i, 0)),
            pl.BlockSpec((BN, D), lambda i: (i, 0)),
        ],
        out_specs=pl.BlockSpec((BN, D), lambda i: (i, 0)),
        out_shape=jax.ShapeDtypeStruct((N, D), x.dtype),
    )(scat, x)
    return out


# speedup vs baseline: 1.0679x; 1.0679x over previous
import jax
import jax.numpy as jnp
from jax import lax
from jax.experimental import pallas as pl
from jax.experimental.pallas import tpu as pltpu
from jax.experimental.pallas import tpu_sc as plsc

N, E, D, DE, H = 10000, 160000, 256, 16, 512
NC, NS = 2, 16
NW = NC * NS
EPW = E // NW
GC = 200
NPW = 320
NPAD = NW * NPW
SCH = 3200
UNR = 4
GB = 128
BE = 1600


def _mesh():
    return plsc.VectorSubcoreMesh(
        core_axis_name="c", subcore_axis_name="s", num_cores=NC, num_subcores=NS
    )


_SC_PARAMS = pltpu.CompilerParams(needs_layout_passes=False, use_tc_tiling_on_sc=False)


def _gather_body(x_hbm, row_hbm, col_hbm, gi_hbm, gj_hbm,
                 ridx, cidx, xi_v, xj_v, sem1, sem2):
    wid = lax.axis_index("s") * NC + lax.axis_index("c")
    base = wid * EPW

    def chunk(k, carry):
        off = base + k * GC
        pltpu.sync_copy(row_hbm.at[pl.ds(off, GC)], ridx)
        pltpu.sync_copy(col_hbm.at[pl.ds(off, GC)], cidx)
        cp1 = pltpu.async_copy(x_hbm.at[ridx], xi_v, sem1)
        cp2 = pltpu.async_copy(x_hbm.at[cidx], xj_v, sem2)
        cp1.wait()
        cp2.wait()
        pltpu.sync_copy(xi_v, gi_hbm.at[pl.ds(off, GC)])
        pltpu.sync_copy(xj_v, gj_hbm.at[pl.ds(off, GC)])
        return carry

    lax.fori_loop(0, EPW // GC, chunk, 0)


def _scatter_body(msg_hbm, col_hbm, zeros_hbm, scat_hbm,
                  cols_v, mid, mdst, rows_v, out_v, sem):
    wid = lax.axis_index("s") * NC + lax.axis_index("c")
    lo = wid * NPW
    hi = lo + NPW
    pltpu.sync_copy(zeros_hbm, out_v)
    zid = jnp.zeros((16,), jnp.int32)

    def iz(j, carry):
        mid[pl.ds(16 * j, 16)] = zid
        return carry

    lax.fori_loop(0, (SCH + 16) // 16, iz, 0)
    iota = lax.iota(jnp.int32, 16)

    def chunk(k, carry):
        cbase = k * SCH
        pltpu.sync_copy(col_hbm.at[pl.ds(cbase, SCH)], cols_v)

        def scan(j, cnt):
            for u in range(UNR):
                o = 16 * (UNR * j + u)
                cv = cols_v[pl.ds(o, 16)]
                m = (cv >= lo) & (cv < hi)
                cs = plsc.cumsum(jnp.where(m, 1, 0))
                idx = cnt + cs - 1
                plsc.store_scatter(mid, [idx], cbase + o + iota, mask=m)
                plsc.store_scatter(mdst, [idx], cv, mask=m)
                cnt = cnt + cs[15]
            return cnt

        cnt = lax.fori_loop(0, SCH // (16 * UNR), scan, 0)

        def bat(g, carry2):
            gb = g * GB
            pltpu.async_copy(msg_hbm.at[mid.at[pl.ds(gb, GB)]], rows_v, sem).wait()

            def rowb(i, carry3):
                r = mdst[pl.ds(i, 16)][0] - lo
                ri = i - gb
                for v in range(16):
                    sl = pl.ds(16 * v, 16)
                    out_v[r, sl] = jnp.maximum(out_v[r, sl], rows_v[ri, sl])
                return carry3

            lax.fori_loop(gb, jnp.minimum(cnt, gb + GB), rowb, 0)
            return carry2

        lax.fori_loop(0, (cnt + GB - 1) // GB, bat, 0)
        return carry

    lax.fori_loop(0, E // SCH, chunk, 0)
    pltpu.sync_copy(out_v, scat_hbm.at[pl.ds(lo, NPW)])


def _mlp_body(xi_ref, xj_ref, ea_ref, w1a, w1b, w1c, w1d, b1_ref, w2, b2_ref,
              msg_ref):
    xi = xi_ref[...]
    xj = xj_ref[...]
    dirn = (2.3094 * jnp.tanh(xj.astype(jnp.float32) - xi.astype(jnp.float32))
            ).astype(jnp.bfloat16)
    ea = ea_ref[...].astype(jnp.bfloat16)

    def dot(a, b):
        return lax.dot_general(a, b, (((1,), (0,)), ((), ())),
                               preferred_element_type=jnp.float32)

    acc = dot(xi, w1a[...]) + dot(xj, w1b[...]) + dot(ea, w1c[...]) + dot(
        dirn, w1d[...])
    h = jnp.maximum(acc + b1_ref[...], 0.0).astype(jnp.bfloat16)
    msg_ref[...] = dot(h, w2[...]) + b2_ref[...]


def _tail(scat_ref, x_ref, o_ref):
    o_ref[...] = 4.0 * jnp.tanh(scat_ref[...] + 0.1 * x_ref[...])


def kernel(x, edge_index, edge_attr, W1, b1, W2, b2):
    row = edge_index[0]
    col = edge_index[1]
    bf = jnp.bfloat16
    x_bf = x.astype(bf)

    gxi, gxj = pl.kernel(
        _gather_body,
        out_type=[
            jax.ShapeDtypeStruct((E, D), bf),
            jax.ShapeDtypeStruct((E, D), bf),
        ],
        mesh=_mesh(),
        compiler_params=_SC_PARAMS,
        scratch_types=[
            pltpu.VMEM((GC,), jnp.int32),
            pltpu.VMEM((GC,), jnp.int32),
            pltpu.VMEM((GC, D), bf),
            pltpu.VMEM((GC, D), bf),
            pltpu.SemaphoreType.DMA,
            pltpu.SemaphoreType.DMA,
        ],
    )(x_bf, row, col)

    W1a = W1[0:D].astype(bf)
    W1b = W1[D:2 * D].astype(bf)
    W1c = W1[2 * D:2 * D + DE].astype(bf)
    W1d = W1[2 * D + DE:].astype(bf)
    W2b = W2.astype(bf)

    msg = pl.pallas_call(
        _mlp_body,
        grid=(E // BE,),
        in_specs=[
            pl.BlockSpec((BE, D), lambda i: (i, 0)),
            pl.BlockSpec((BE, D), lambda i: (i, 0)),
            pl.BlockSpec((BE, DE), lambda i: (i, 0)),
            pl.BlockSpec((D, H), lambda i: (0, 0)),
            pl.BlockSpec((D, H), lambda i: (0, 0)),
            pl.BlockSpec((DE, H), lambda i: (0, 0)),
            pl.BlockSpec((D, H), lambda i: (0, 0)),
            pl.BlockSpec((1, H), lambda i: (0, 0)),
            pl.BlockSpec((H, D), lambda i: (0, 0)),
            pl.BlockSpec((1, D), lambda i: (0, 0)),
        ],
        out_specs=pl.BlockSpec((BE, D), lambda i: (i, 0)),
        out_shape=jax.ShapeDtypeStruct((E, D), jnp.float32),
    )(gxi, gxj, edge_attr, W1a, W1b, W1c, W1d, b1.reshape(1, H), W2b,
      b2.reshape(1, D))

    zeros = jnp.zeros((NPW, D), jnp.float32)
    scat = pl.kernel(
        _scatter_body,
        out_type=jax.ShapeDtypeStruct((NPAD, D), jnp.float32),
        mesh=_mesh(),
        compiler_params=_SC_PARAMS,
        scratch_types=[
            pltpu.VMEM((SCH,), jnp.int32),
            pltpu.VMEM((SCH + 16,), jnp.int32),
            pltpu.VMEM((SCH + 16,), jnp.int32),
            pltpu.VMEM((GB, D), jnp.float32),
            pltpu.VMEM((NPW, D), jnp.float32),
            pltpu.SemaphoreType.DMA,
        ],
    )(msg, col, zeros)

    BN = 1000
    out = pl.pallas_call(
        _tail,
        grid=(N // BN,),
        in_specs=[
            pl.BlockSpec((BN, D), lambda i: (i, 0)),
            pl.BlockSpec((BN, D), lambda i: (i, 0)),
        ],
        out_specs=pl.BlockSpec((BN, D), lambda i: (i, 0)),
        out_shape=jax.ShapeDtypeStruct((N, D), x.dtype),
    )(scat, x)
    return out


# speedup vs baseline: 1.0893x; 1.0201x over previous
import functools

import jax
import jax.numpy as jnp
from jax import lax
from jax.experimental import pallas as pl
from jax.experimental.pallas import tpu as pltpu
from jax.experimental.pallas import tpu_sc as plsc

N, E, D, DE, H = 10000, 160000, 256, 16, 512
NC, NS = 2, 16
NW = NC * NS
GC = 200
NPW = 320
NPAD = NW * NPW
SCH = 3200
UNR = 4
GB = 128
BE = 1600
EH0, EH1 = 83200, 76800


def _mesh():
    return plsc.VectorSubcoreMesh(
        core_axis_name="c", subcore_axis_name="s", num_cores=NC, num_subcores=NS
    )


_SC_GATHER_PARAMS = pltpu.CompilerParams(
    needs_layout_passes=False, use_tc_tiling_on_sc=False)
_SC_SCATTER_PARAMS = pltpu.CompilerParams(needs_layout_passes=False)


def _gather_body(ecount, x_hbm, row_hbm, col_hbm, gi_hbm, gj_hbm,
                 ridx, cidx, xi_v, xj_v, sem1, sem2):
    epw = ecount // NW
    wid = lax.axis_index("s") * NC + lax.axis_index("c")
    base = wid * epw

    def chunk(k, carry):
        off = base + k * GC
        pltpu.sync_copy(row_hbm.at[pl.ds(off, GC)], ridx)
        pltpu.sync_copy(col_hbm.at[pl.ds(off, GC)], cidx)
        cp1 = pltpu.async_copy(x_hbm.at[ridx], xi_v, sem1)
        cp2 = pltpu.async_copy(x_hbm.at[cidx], xj_v, sem2)
        cp1.wait()
        cp2.wait()
        pltpu.sync_copy(xi_v, gi_hbm.at[pl.ds(off, GC)])
        pltpu.sync_copy(xj_v, gj_hbm.at[pl.ds(off, GC)])
        return carry

    lax.fori_loop(0, epw // GC, chunk, 0)


def _scatter_body(ecount, msg_hbm, col_hbm, init_hbm, scat_hbm,
                  cols_v, mid, mdst, rows_v, out_v, sem):
    wid = lax.axis_index("s") * NC + lax.axis_index("c")
    lo = wid * NPW
    hi = lo + NPW
    pltpu.sync_copy(init_hbm.at[pl.ds(lo, NPW)], out_v)
    zid = jnp.zeros((16,), jnp.int32)

    def iz(j, carry):
        mid[pl.ds(16 * j, 16)] = zid
        return carry

    lax.fori_loop(0, (SCH + 16) // 16, iz, 0)
    iota = lax.iota(jnp.int32, 16)

    def chunk(k, carry):
        cbase = k * SCH
        pltpu.sync_copy(col_hbm.at[pl.ds(cbase, SCH)], cols_v)

        def scan(j, cnt):
            for u in range(UNR):
                o = 16 * (UNR * j + u)
                cv = cols_v[pl.ds(o, 16)]
                m = (cv >= lo) & (cv < hi)
                cs = plsc.cumsum(jnp.where(m, 1, 0))
                idx = cnt + cs - 1
                plsc.store_scatter(mid, [idx], cbase + o + iota, mask=m)
                plsc.store_scatter(mdst, [idx], cv, mask=m)
                cnt = cnt + cs[15]
            return cnt

        cnt = lax.fori_loop(0, SCH // (16 * UNR), scan, 0)

        def bat(g, carry2):
            gb = g * GB
            pltpu.async_copy(msg_hbm.at[mid.at[pl.ds(gb, GB)]], rows_v, sem).wait()

            def rowb(i, carry3):
                r = mdst[pl.ds(i, 16)][0] - lo
                ri = i - gb
                for v in range(16):
                    sl = pl.ds(16 * v, 16)
                    out_v[r, sl] = jnp.maximum(out_v[r, sl], rows_v[ri, sl])
                return carry3

            lax.fori_loop(gb, jnp.minimum(cnt, gb + GB), rowb, 0)
            return carry2

        lax.fori_loop(0, (cnt + GB - 1) // GB, bat, 0)
        return carry

    lax.fori_loop(0, ecount // SCH, chunk, 0)
    pltpu.sync_copy(out_v, scat_hbm.at[pl.ds(lo, NPW)])


def _mlp_body(xi_ref, xj_ref, ea_ref, w1a, w1b, w1c, w1d, b1_ref, w2, b2_ref,
              msg_ref):
    xi = xi_ref[...]
    xj = xj_ref[...]
    dirn = (2.3094 * jnp.tanh(xj.astype(jnp.float32) - xi.astype(jnp.float32))
            ).astype(jnp.bfloat16)
    ea = ea_ref[...].astype(jnp.bfloat16)

    def dot(a, b):
        return lax.dot_general(a, b, (((1,), (0,)), ((), ())),
                               preferred_element_type=jnp.float32)

    acc = dot(xi, w1a[...]) + dot(xj, w1b[...]) + dot(ea, w1c[...]) + dot(
        dirn, w1d[...])
    h = jnp.maximum(acc + b1_ref[...], 0.0).astype(jnp.bfloat16)
    msg_ref[...] = dot(h, w2[...]) + b2_ref[...]


def _tail(scat_ref, x_ref, o_ref):
    o_ref[...] = 4.0 * jnp.tanh(scat_ref[...] + 0.1 * x_ref[...])


def _gather_half(x_bf, row_h, col_h, ecount):
    return pl.kernel(
        functools.partial(_gather_body, ecount),
        out_type=[
            jax.ShapeDtypeStruct((ecount, D), jnp.bfloat16),
            jax.ShapeDtypeStruct((ecount, D), jnp.bfloat16),
        ],
        mesh=_mesh(),
        compiler_params=_SC_GATHER_PARAMS,
        scratch_types=[
            pltpu.VMEM((GC,), jnp.int32),
            pltpu.VMEM((GC,), jnp.int32),
            pltpu.VMEM((GC, D), jnp.bfloat16),
            pltpu.VMEM((GC, D), jnp.bfloat16),
            pltpu.SemaphoreType.DMA,
            pltpu.SemaphoreType.DMA,
        ],
    )(x_bf, row_h, col_h)


def _mlp_half(gxi, gxj, ea_h, weights, ecount):
    W1a, W1b, W1c, W1d, b1r, W2b, b2r = weights
    return pl.pallas_call(
        _mlp_body,
        grid=(ecount // BE,),
        in_specs=[
            pl.BlockSpec((BE, D), lambda i: (i, 0)),
            pl.BlockSpec((BE, D), lambda i: (i, 0)),
            pl.BlockSpec((BE, DE), lambda i: (i, 0)),
            pl.BlockSpec((D, H), lambda i: (0, 0)),
            pl.BlockSpec((D, H), lambda i: (0, 0)),
            pl.BlockSpec((DE, H), lambda i: (0, 0)),
            pl.BlockSpec((D, H), lambda i: (0, 0)),
            pl.BlockSpec((1, H), lambda i: (0, 0)),
            pl.BlockSpec((H, D), lambda i: (0, 0)),
            pl.BlockSpec((1, D), lambda i: (0, 0)),
        ],
        out_specs=pl.BlockSpec((BE, D), lambda i: (i, 0)),
        out_shape=jax.ShapeDtypeStruct((ecount, D), jnp.float32),
    )(gxi, gxj, ea_h, W1a, W1b, W1c, W1d, b1r, W2b, b2r)


def _scatter_half(msg_h, col_h, init, ecount):
    return pl.kernel(
        functools.partial(_scatter_body, ecount),
        out_type=jax.ShapeDtypeStruct((NPAD, D), jnp.float32),
        mesh=_mesh(),
        compiler_params=_SC_SCATTER_PARAMS,
        scratch_types=[
            pltpu.VMEM((SCH,), jnp.int32),
            pltpu.VMEM((SCH + 16,), jnp.int32),
            pltpu.VMEM((SCH + 16,), jnp.int32),
            pltpu.VMEM((GB, D), jnp.float32),
            pltpu.VMEM((NPW, D), jnp.float32),
            pltpu.SemaphoreType.DMA,
        ],
    )(msg_h, col_h, init)


def kernel(x, edge_index, edge_attr, W1, b1, W2, b2):
    row = edge_index[0]
    col = edge_index[1]
    bf = jnp.bfloat16
    x_bf = x.astype(bf)

    W1a = W1[0:D].astype(bf)
    W1b = W1[D:2 * D].astype(bf)
    W1c = W1[2 * D:2 * D + DE].astype(bf)
    W1d = W1[2 * D + DE:].astype(bf)
    weights = (W1a, W1b, W1c, W1d, b1.reshape(1, H), W2.astype(bf),
               b2.reshape(1, D))

    bounds = ((0, EH0), (EH0, EH1))
    scat = jnp.zeros((NPAD, D), jnp.float32)
    for start, ecount in bounds:
        row_h = lax.slice(row, (start,), (start + ecount,))
        col_h = lax.slice(col, (start,), (start + ecount,))
        ea_h = lax.slice(edge_attr, (start, 0), (start + ecount, DE))
        gxi, gxj = _gather_half(x_bf, row_h, col_h, ecount)
        msg_h = _mlp_half(gxi, gxj, ea_h, weights, ecount)
        scat = _scatter_half(msg_h, col_h, scat, ecount)

    BN = 1000
    out = pl.pallas_call(
        _tail,
        grid=(N // BN,),
        in_specs=[
            pl.BlockSpec((BN, D), lambda i: (i, 0)),
            pl.BlockSpec((BN, D), lambda i: (i, 0)),
        ],
        out_specs=pl.BlockSpec((BN, D), lambda i: (i, 0)),
        out_shape=jax.ShapeDtypeStruct((N, D), x.dtype),
    )(scat, x)
    return out
